# baseline (speedup 1.0000x reference)
import functools

import jax
import jax.numpy as jnp
from jax import lax
from jax.experimental import pallas as pl
from jax.experimental.pallas import tpu as pltpu
from jax.experimental.pallas import tpu_sc as plsc

_D = 128
_BN = 4096
_T = 50
_NC, _NS = 2, 16
_NW = _NC * _NS
_BPW = _BN // _NW
_CB = 2
_C = _CB * _T
_K = 4
_NCH = _BPW // _CB
_NGRP = _NCH // _K


@functools.partial(
    pl.kernel,
    out_type=jax.ShapeDtypeStruct((_BN, _T, _D), jnp.float32),
    mesh=plsc.VectorSubcoreMesh(core_axis_name="c", subcore_axis_name="s"),
    scratch_types=[
        pltpu.VMEM((_NCH, _C), jnp.int32),
        pltpu.VMEM((2, _K, _C, _D), jnp.float32),
    ] + [pltpu.SemaphoreType.DMA] * (4 * _K),
    compiler_params=pltpu.CompilerParams(use_tc_tiling_on_sc=True),
)
def _emb_gather(table_hbm, idx_hbm, out_hbm, idx_v, bufs, *sems):
    gsems = (sems[0:_K], sems[_K:2 * _K])
    ssems = (sems[2 * _K:3 * _K], sems[3 * _K:4 * _K])
    wid = lax.axis_index("s") * _NC + lax.axis_index("c")
    base = wid * _BPW
    pltpu.sync_copy(idx_hbm.at[wid], idx_v)

    def process(g, h_cur, h_nxt):
        for b in range(_K):
            wb = base + (g * _K + b) * _CB
            pltpu.make_async_copy(table_hbm.at[idx_v.at[0]],
                                  bufs.at[h_cur, b], gsems[h_cur][b]).wait()
            for r in range(_CB):
                pltpu.async_copy(bufs.at[h_cur, b, pl.ds(r * _T, _T)],
                                 out_hbm.at[wb + r], ssems[h_cur][b])

        @pl.when(g + 1 < _NGRP)
        def _issue():
            for b in range(_K):
                @pl.when(g > 0)
                def _drain():
                    for r in range(_CB):
                        pltpu.make_async_copy(
                            bufs.at[h_nxt, b, pl.ds(r * _T, _T)],
                            out_hbm.at[base], ssems[h_nxt][b]).wait()
                jn = (g + 1) * _K + b
                pltpu.async_copy(table_hbm.at[idx_v.at[jn]],
                                 bufs.at[h_nxt, b], gsems[h_nxt][b])

    for b in range(_K):
        pltpu.async_copy(table_hbm.at[idx_v.at[b]], bufs.at[0, b],
                         gsems[0][b])

    def loop_body(gg, carry):
        process(2 * gg, 0, 1)
        process(2 * gg + 1, 1, 0)
        return carry

    lax.fori_loop(0, _NGRP // 2, loop_body, 0)

    for h in range(2):
        for b in range(_K):
            for r in range(_CB):
                pltpu.make_async_copy(bufs.at[h, b, pl.ds(r * _T, _T)],
                                      out_hbm.at[base], ssems[h][b]).wait()


def _lin_body(x_ref, w_ref, o_ref):
    o_ref[:] = lax.dot_general(
        x_ref[:], w_ref[:], (((1,), (1,)), ((), ())),
        preferred_element_type=jnp.float32)


def _linear(x, w):
    return pl.pallas_call(
        _lin_body,
        out_shape=jax.ShapeDtypeStruct(x.shape, jnp.float32),
        grid=(8,),
        in_specs=[
            pl.BlockSpec((x.shape[0] // 8, _D), lambda i: (i, 0)),
            pl.BlockSpec((_D, _D), lambda i: (0, 0)),
        ],
        out_specs=pl.BlockSpec((x.shape[0] // 8, _D), lambda i: (i, 0)),
    )(x, w)


def kernel(token_ids, dense_feat, embedding_weight, linear_weight):
    idx = token_ids.astype(jnp.int32).reshape(_NW, _NCH, _C)
    emb_out = _emb_gather(embedding_weight, idx)
    lin_out = _linear(dense_feat.astype(jnp.float32), linear_weight)
    return emb_out, lin_out


# speedup vs baseline: 1.7182x; 1.7182x over previous
import functools

import jax
import jax.numpy as jnp
from jax import lax
from jax.experimental import pallas as pl
from jax.experimental.pallas import tpu as pltpu
from jax.experimental.pallas import tpu_sc as plsc

_D = 128
_BN = 4096
_T = 50
_B = _BN * _T
_NC, _NS = 2, 16
_NW = _NC * _NS
_PER_W = _B // _NW
_C = 80
_K = 4
_NCH = _PER_W // _C
_NGRP = _NCH // _K


@functools.partial(
    pl.kernel,
    out_type=jax.ShapeDtypeStruct((_B, _D), jnp.float32),
    mesh=plsc.VectorSubcoreMesh(core_axis_name="c", subcore_axis_name="s"),
    scratch_types=[
        pltpu.VMEM((_NCH, _C), jnp.int32),
        pltpu.VMEM((2, _K, _C, _D), jnp.float32),
    ] + [pltpu.SemaphoreType.DMA] * (4 * _K),
)
def _emb_gather(table_hbm, idx_hbm, out_hbm, idx_v, bufs, *sems):
    gsems = (sems[0:_K], sems[_K:2 * _K])
    ssems = (sems[2 * _K:3 * _K], sems[3 * _K:4 * _K])
    wid = lax.axis_index("s") * _NC + lax.axis_index("c")
    base = wid * _PER_W
    pltpu.sync_copy(idx_hbm.at[wid], idx_v)

    def process(g, h_cur, h_nxt):
        for b in range(_K):
            j = g * _K + b
            pltpu.make_async_copy(table_hbm.at[idx_v.at[0]],
                                  bufs.at[h_cur, b], gsems[h_cur][b]).wait()
            pltpu.async_copy(bufs.at[h_cur, b],
                             out_hbm.at[pl.ds(base + j * _C, _C)],
                             ssems[h_cur][b])

        @pl.when(g + 1 < _NGRP)
        def _issue():
            for b in range(_K):
                @pl.when(g > 0)
                def _drain():
                    pltpu.make_async_copy(bufs.at[h_nxt, b],
                                          out_hbm.at[pl.ds(base, _C)],
                                          ssems[h_nxt][b]).wait()
                jn = (g + 1) * _K + b
                pltpu.async_copy(table_hbm.at[idx_v.at[jn]],
                                 bufs.at[h_nxt, b], gsems[h_nxt][b])

    for b in range(_K):
        pltpu.async_copy(table_hbm.at[idx_v.at[b]], bufs.at[0, b],
                         gsems[0][b])

    def loop_body(gg, carry):
        process(2 * gg, 0, 1)
        process(2 * gg + 1, 1, 0)
        return carry

    lax.fori_loop(0, _NGRP // 2, loop_body, 0)

    for h in range(2):
        for b in range(_K):
            pltpu.make_async_copy(bufs.at[h, b],
                                  out_hbm.at[pl.ds(base, _C)],
                                  ssems[h][b]).wait()


def _lin_body(x_ref, w_ref, o_ref):
    o_ref[:] = lax.dot_general(
        x_ref[:], w_ref[:], (((1,), (1,)), ((), ())),
        preferred_element_type=jnp.float32)


def _linear(x, w):
    return pl.pallas_call(
        _lin_body,
        out_shape=jax.ShapeDtypeStruct(x.shape, jnp.float32),
        grid=(8,),
        in_specs=[
            pl.BlockSpec((x.shape[0] // 8, _D), lambda i: (i, 0)),
            pl.BlockSpec((_D, _D), lambda i: (0, 0)),
        ],
        out_specs=pl.BlockSpec((x.shape[0] // 8, _D), lambda i: (i, 0)),
    )(x, w)


def kernel(token_ids, dense_feat, embedding_weight, linear_weight):
    idx = token_ids.astype(jnp.int32).T.reshape(_NW, _NCH, _C)
    flat = _emb_gather(embedding_weight, idx)
    emb_out = flat.reshape(_T, _BN, _D).transpose(1, 0, 2)
    lin_out = _linear(dense_feat.astype(jnp.float32), linear_weight)
    return emb_out, lin_out


# speedup vs baseline: 1.7363x; 1.0105x over previous
import functools

import jax
import jax.numpy as jnp
from jax import lax
from jax.experimental import pallas as pl
from jax.experimental.pallas import tpu as pltpu
from jax.experimental.pallas import tpu_sc as plsc

_D = 128
_BN = 4096
_T = 50
_B = _BN * _T
_NC, _NS = 2, 16
_NW = _NC * _NS
_PER_W = _B // _NW
_C = 80
_K = 4
_NCH = _PER_W // _C
_NGRP = _NCH // _K


@functools.partial(
    pl.kernel,
    out_type=jax.ShapeDtypeStruct((_B, _D), jnp.float32),
    mesh=plsc.VectorSubcoreMesh(core_axis_name="c", subcore_axis_name="s"),
    scratch_types=[
        pltpu.VMEM((_NCH, _C), jnp.int32),
        pltpu.VMEM((2, _K * _C, _D), jnp.float32),
    ] + [pltpu.SemaphoreType.DMA] * (2 * _K + 2),
)
def _emb_gather(table_hbm, idx_hbm, out_hbm, idx_v, bufs, *sems):
    gsems = (sems[0:_K], sems[_K:2 * _K])
    ssems = sems[2 * _K:2 * _K + 2]
    wid = lax.axis_index("s") * _NC + lax.axis_index("c")
    base = wid * _PER_W
    pltpu.sync_copy(idx_hbm.at[wid], idx_v)

    def process(g, h_cur, h_nxt):
        for b in range(_K):
            pltpu.make_async_copy(table_hbm.at[idx_v.at[0]],
                                  bufs.at[h_cur, pl.ds(b * _C, _C)],
                                  gsems[h_cur][b]).wait()
        pltpu.async_copy(bufs.at[h_cur],
                         out_hbm.at[pl.ds(base + g * _K * _C, _K * _C)],
                         ssems[h_cur])

        @pl.when(g + 1 < _NGRP)
        def _issue():
            @pl.when(g > 0)
            def _drain():
                pltpu.make_async_copy(bufs.at[h_nxt],
                                      out_hbm.at[pl.ds(base, _K * _C)],
                                      ssems[h_nxt]).wait()
            for b in range(_K):
                jn = (g + 1) * _K + b
                pltpu.async_copy(table_hbm.at[idx_v.at[jn]],
                                 bufs.at[h_nxt, pl.ds(b * _C, _C)],
                                 gsems[h_nxt][b])

    for b in range(_K):
        pltpu.async_copy(table_hbm.at[idx_v.at[b]],
                         bufs.at[0, pl.ds(b * _C, _C)], gsems[0][b])

    def loop_body(gg, carry):
        process(2 * gg, 0, 1)
        process(2 * gg + 1, 1, 0)
        return carry

    lax.fori_loop(0, _NGRP // 2, loop_body, 0)

    for h in range(2):
        pltpu.make_async_copy(bufs.at[h],
                              out_hbm.at[pl.ds(base, _K * _C)],
                              ssems[h]).wait()


def _lin_body(x_ref, w_ref, o_ref):
    o_ref[:] = lax.dot_general(
        x_ref[:], w_ref[:], (((1,), (1,)), ((), ())),
        preferred_element_type=jnp.float32)


def _linear(x, w):
    return pl.pallas_call(
        _lin_body,
        out_shape=jax.ShapeDtypeStruct(x.shape, jnp.float32),
        grid=(8,),
        in_specs=[
            pl.BlockSpec((x.shape[0] // 8, _D), lambda i: (i, 0)),
            pl.BlockSpec((_D, _D), lambda i: (0, 0)),
        ],
        out_specs=pl.BlockSpec((x.shape[0] // 8, _D), lambda i: (i, 0)),
    )(x, w)


def kernel(token_ids, dense_feat, embedding_weight, linear_weight):
    idx = token_ids.astype(jnp.int32).T.reshape(_NW, _NCH, _C)
    flat = _emb_gather(embedding_weight, idx)
    emb_out = flat.reshape(_T, _BN, _D).transpose(1, 0, 2)
    lin_out = _linear(dense_feat.astype(jnp.float32), linear_weight)
    return emb_out, lin_out
